# baseline (speedup 1.0000x reference)
import functools

import jax
import jax.numpy as jnp
from jax import lax
from jax.experimental import pallas as pl
from jax.experimental.pallas import tpu as pltpu
from jax.experimental.pallas import tpu_sc as plsc

NODE = 10000
D = 128
E = 320000
ALPHA = 0.5
DP = 144
NC = 2
NS = 16
NW = NC * NS
EPW = E // NW
B = 80
NCHUNK = EPW // B
NP = 10240
ROWS_PER_TILE = NP // NS
BLK = 1000
NBLK = NODE // BLK


def _prologue_body(f_ref, ha_ref, da_ref, gq_ref):
    F = jnp.tanh(f_ref[...])
    w = ha_ref[...] - ALPHA * da_ref[...]
    p = lax.dot_general(F, w, (((1,), (0,)), ((), ())),
                        preferred_element_type=jnp.float32)
    q = jnp.exp(p)
    gq_ref[...] = jnp.concatenate(
        [F * q, q, jnp.zeros((BLK, DP - D - 1), jnp.float32)], axis=1)


def _epilogue_body(h0_ref, h1_ref, o_ref):
    h = h0_ref[...] + h1_ref[...]
    q = h[:, D:D + 1]
    o_ref[...] = jnp.tanh(jnp.where(q > 0, h[:, :D] / q, 0.0))


def _sc_body(gq_hbm, row3_hbm, col3_hbm, zero_hbm, out0_hbm, out1_hbm,
             colv, rowv0, rowv1, rows0, rows1, acc, gsem, rsem):
    cid = lax.axis_index("c")
    sid = lax.axis_index("s")
    wid = cid * NS + sid
    zbase = sid * ROWS_PER_TILE
    pltpu.sync_copy(zero_hbm.at[pl.ds(0, ROWS_PER_TILE)],
                    acc.at[pl.ds(zbase, ROWS_PER_TILE)])
    pltpu.sync_copy(col3_hbm.at[wid], colv)
    plsc.subcore_barrier()

    def g_start(c, buf):
        pltpu.async_copy(gq_hbm.at[colv.at[c]], buf, gsem)

    def g_wait(c, buf):
        pltpu.make_async_copy(gq_hbm.at[colv.at[c]], buf, gsem).wait()

    def r_start(c, rbuf):
        pltpu.async_copy(row3_hbm.at[wid, c], rbuf, rsem)

    def r_wait(c, rbuf):
        pltpu.make_async_copy(row3_hbm.at[wid, c], rbuf, rsem).wait()

    def s_add(buf, rbuf):
        pltpu.sync_copy(buf, acc.at[rbuf], add=True)

    g_start(0, rows0)
    r_start(0, rowv0)

    def body(j, carry):
        c0 = 2 * j
        g_wait(c0, rows0)
        r_wait(c0, rowv0)
        g_start(c0 + 1, rows1)
        r_start(c0 + 1, rowv1)
        s_add(rows0, rowv0)
        g_wait(c0 + 1, rows1)
        r_wait(c0 + 1, rowv1)
        g_start(c0 + 2, rows0)
        r_start(c0 + 2, rowv0)
        s_add(rows1, rowv1)
        return carry

    lax.fori_loop(0, (NCHUNK - 1) // 2, body, 0)
    g_wait(NCHUNK - 1, rows0)
    r_wait(NCHUNK - 1, rowv0)
    s_add(rows0, rowv0)
    plsc.subcore_barrier()

    src = acc.at[pl.ds(zbase, ROWS_PER_TILE)]

    @pl.when(cid == 0)
    def _():
        pltpu.sync_copy(src, out0_hbm.at[pl.ds(zbase, ROWS_PER_TILE)])

    @pl.when(cid == 1)
    def _():
        pltpu.sync_copy(src, out1_hbm.at[pl.ds(zbase, ROWS_PER_TILE)])


def kernel(features, adj_nei, high_att_0, diff_att_0):
    gq = pl.pallas_call(
        _prologue_body,
        grid=(NBLK,),
        in_specs=[
            pl.BlockSpec((BLK, D), lambda i: (i, 0)),
            pl.BlockSpec((D, 1), lambda i: (0, 0)),
            pl.BlockSpec((D, 1), lambda i: (0, 0)),
        ],
        out_specs=pl.BlockSpec((BLK, DP), lambda i: (i, 0)),
        out_shape=jax.ShapeDtypeStruct((NODE, DP), jnp.float32),
    )(features, high_att_0, diff_att_0)

    row3 = adj_nei[0].reshape(NW, NCHUNK, B)
    col3 = adj_nei[1].reshape(NW, NCHUNK, B)
    zeros = jnp.zeros((ROWS_PER_TILE, DP), jnp.float32)

    sc_fn = functools.partial(
        pl.kernel,
        mesh=plsc.VectorSubcoreMesh(core_axis_name="c", subcore_axis_name="s"),
        out_type=(jax.ShapeDtypeStruct((NP, DP), jnp.float32),
                  jax.ShapeDtypeStruct((NP, DP), jnp.float32)),
        scratch_types=[
            pltpu.VMEM((NCHUNK, B), jnp.int32),
            pltpu.VMEM((B,), jnp.int32),
            pltpu.VMEM((B,), jnp.int32),
            pltpu.VMEM((B, DP), jnp.float32),
            pltpu.VMEM((B, DP), jnp.float32),
            pltpu.VMEM_SHARED((NP, DP), jnp.float32),
            pltpu.SemaphoreType.DMA,
            pltpu.SemaphoreType.DMA,
        ],
        compiler_params=pltpu.CompilerParams(use_tc_tiling_on_sc=False),
    )(_sc_body)
    hp0, hp1 = sc_fn(gq, row3, col3, zeros)

    out = pl.pallas_call(
        _epilogue_body,
        grid=(NBLK,),
        in_specs=[
            pl.BlockSpec((BLK, DP), lambda i: (i, 0)),
            pl.BlockSpec((BLK, DP), lambda i: (i, 0)),
        ],
        out_specs=pl.BlockSpec((BLK, D), lambda i: (i, 0)),
        out_shape=jax.ShapeDtypeStruct((NODE, D), jnp.float32),
    )(hp0, hp1)
    return out


# speedup vs baseline: 1.2422x; 1.2422x over previous
import functools

import jax
import jax.numpy as jnp
from jax import lax
from jax.experimental import pallas as pl
from jax.experimental.pallas import tpu as pltpu
from jax.experimental.pallas import tpu_sc as plsc

NODE = 10000
D = 128
E = 320000
ALPHA = 0.5
DP = 144
NC = 2
NS = 16
NW = NC * NS
B = 64
NCHUNK = 160
EPW = NCHUNK * B
EPAD = NW * EPW - E
NP = 10240
ROWS_PER_TILE = NP // NS
BLK = 1000
NBLK = NODE // BLK


def _prologue_body(f_ref, ha_ref, da_ref, gq_ref):
    F = jnp.tanh(f_ref[...])
    w = ha_ref[...] - ALPHA * da_ref[...]
    p = lax.dot_general(F, w, (((1,), (0,)), ((), ())),
                        preferred_element_type=jnp.float32)
    q = jnp.exp(p)
    gq_ref[...] = jnp.concatenate(
        [F * q, q, jnp.zeros((BLK, DP - D - 1), jnp.float32)], axis=1)


def _epilogue_body(h0_ref, h1_ref, o_ref):
    h = h0_ref[...] + h1_ref[...]
    q = h[:, D:D + 1]
    o_ref[...] = jnp.tanh(jnp.where(q > 0, h[:, :D] / q, 0.0))


def _sc_body(gq_hbm, row3_hbm, col3_hbm, zero_hbm, out0_hbm, out1_hbm,
             colv, rowv0, rowv1, rowv2, rows0, rows1, rows2, acc,
             gs0, gs1, gs2, ss0, ss1, ss2, rs0, rs1, rs2):
    cid = lax.axis_index("c")
    sid = lax.axis_index("s")
    wid = cid * NS + sid
    zbase = sid * ROWS_PER_TILE
    pltpu.sync_copy(zero_hbm.at[pl.ds(0, ROWS_PER_TILE)],
                    acc.at[pl.ds(zbase, ROWS_PER_TILE)])
    pltpu.sync_copy(col3_hbm.at[wid], colv)
    plsc.subcore_barrier()

    rows = (rows0, rows1, rows2)
    rowv = (rowv0, rowv1, rowv2)
    gsem = (gs0, gs1, gs2)
    ssem = (ss0, ss1, ss2)
    rsem = (rs0, rs1, rs2)

    def g_start(c, a):
        pltpu.async_copy(gq_hbm.at[colv.at[c]], rows[a], gsem[a])

    def g_wait(c, a):
        pltpu.make_async_copy(gq_hbm.at[colv.at[c]], rows[a], gsem[a]).wait()

    def r_start(c, a):
        pltpu.async_copy(row3_hbm.at[wid, c], rowv[a], rsem[a])

    def r_wait(c, a):
        pltpu.make_async_copy(row3_hbm.at[wid, c], rowv[a], rsem[a]).wait()

    def s_start(a):
        pltpu.async_copy(rows[a], acc.at[rowv[a]], ssem[a], add=True)

    def s_wait(a):
        pltpu.make_async_copy(rows[a], acc.at[rowv[a]], ssem[a]).wait()

    g_start(0, 0)
    r_start(0, 0)
    g_start(1, 1)
    r_start(1, 1)

    def body(j, carry):
        c0 = 3 * j
        for k in range(3):
            c = c0 + k
            a = k % 3
            prev = (k + 2) % 3
            g_wait(c, a)
            r_wait(c, a)
            s_start(a)
            if k == 0:
                @pl.when(j > 0)
                def _():
                    s_wait(prev)
                g_start(c + 2, prev)
                r_start(c + 2, prev)
            elif k == 1:
                s_wait(prev)
                g_start(c + 2, prev)
                r_start(c + 2, prev)
            else:
                s_wait(prev)

                @pl.when(j < (NCHUNK - 1) // 3 - 1)
                def _():
                    g_start(c + 2, prev)
                    r_start(c + 2, prev)
        return carry

    lax.fori_loop(0, (NCHUNK - 1) // 3, body, 0)
    c_last = NCHUNK - 1
    g_wait(c_last, c_last % 3)
    r_wait(c_last, c_last % 3)
    s_start(c_last % 3)
    s_wait((c_last + 2) % 3)
    s_wait(c_last % 3)
    plsc.subcore_barrier()

    src = acc.at[pl.ds(zbase, ROWS_PER_TILE)]

    @pl.when(cid == 0)
    def _():
        pltpu.sync_copy(src, out0_hbm.at[pl.ds(zbase, ROWS_PER_TILE)])

    @pl.when(cid == 1)
    def _():
        pltpu.sync_copy(src, out1_hbm.at[pl.ds(zbase, ROWS_PER_TILE)])


def kernel(features, adj_nei, high_att_0, diff_att_0):
    gq = pl.pallas_call(
        _prologue_body,
        grid=(NBLK,),
        in_specs=[
            pl.BlockSpec((BLK, D), lambda i: (i, 0)),
            pl.BlockSpec((D, 1), lambda i: (0, 0)),
            pl.BlockSpec((D, 1), lambda i: (0, 0)),
        ],
        out_specs=pl.BlockSpec((BLK, DP), lambda i: (i, 0)),
        out_shape=jax.ShapeDtypeStruct((NODE, DP), jnp.float32),
    )(features, high_att_0, diff_att_0)

    pad_i = jnp.arange(EPAD, dtype=jnp.int32)
    row3 = jnp.concatenate(
        [adj_nei[0], NODE + pad_i % (NP - NODE)]).reshape(NW, NCHUNK, B)
    col3 = jnp.concatenate(
        [adj_nei[1], pad_i % NODE]).reshape(NW, NCHUNK, B)
    zeros = jnp.zeros((ROWS_PER_TILE, DP), jnp.float32)

    sc_fn = functools.partial(
        pl.kernel,
        mesh=plsc.VectorSubcoreMesh(core_axis_name="c", subcore_axis_name="s"),
        out_type=(jax.ShapeDtypeStruct((NP, DP), jnp.float32),
                  jax.ShapeDtypeStruct((NP, DP), jnp.float32)),
        scratch_types=(
            [pltpu.VMEM((NCHUNK, B), jnp.int32)]
            + [pltpu.VMEM((B,), jnp.int32) for _ in range(3)]
            + [pltpu.VMEM((B, DP), jnp.float32) for _ in range(3)]
            + [pltpu.VMEM_SHARED((NP, DP), jnp.float32)]
            + [pltpu.SemaphoreType.DMA for _ in range(9)]
        ),
        compiler_params=pltpu.CompilerParams(use_tc_tiling_on_sc=False),
    )(_sc_body)
    hp0, hp1 = sc_fn(gq, row3, col3, zeros)

    out = pl.pallas_call(
        _epilogue_body,
        grid=(NBLK,),
        in_specs=[
            pl.BlockSpec((BLK, DP), lambda i: (i, 0)),
            pl.BlockSpec((BLK, DP), lambda i: (i, 0)),
        ],
        out_specs=pl.BlockSpec((BLK, D), lambda i: (i, 0)),
        out_shape=jax.ShapeDtypeStruct((NODE, D), jnp.float32),
    )(hp0, hp1)
    return out


# speedup vs baseline: 1.5175x; 1.2216x over previous
import functools

import jax
import jax.numpy as jnp
from jax import lax
from jax.experimental import pallas as pl
from jax.experimental.pallas import tpu as pltpu
from jax.experimental.pallas import tpu_sc as plsc

NODE = 10000
D = 128
E = 320000
ALPHA = 0.5
NC = 2
NS = 16
NW = NC * NS
B = 64
NCHUNK = 160
EPW = NCHUNK * B
EPAD = NW * EPW - E
NP = 10240
ROWS_PER_TILE = NP // NS


def _prologue_body(f_ref, ha_ref, da_ref, g_ref, qt_ref):
    F = jnp.tanh(f_ref[...])
    w = ha_ref[...] - ALPHA * da_ref[...]
    p = lax.dot_general(F, w, (((1,), (0,)), ((), ())),
                        preferred_element_type=jnp.float32)
    q = jnp.exp(p)
    g_ref[...] = F * q
    qt_ref[...] = lax.transpose(q, (1, 0))


def _epilogue_body(h0_ref, h1_ref, qp_ref, o_ref):
    h = h0_ref[...] + h1_ref[...]
    qs = jnp.sum(qp_ref[...], axis=0)
    qc = qs[:NODE, None]
    o_ref[...] = jnp.tanh(jnp.where(qc > 0, h[:NODE] / qc, 0.0))


def _sc_body(g_hbm, q_hbm, row3_hbm, col3_hbm, zero_hbm, zq_hbm,
             out0_hbm, out1_hbm, qp_hbm,
             qtab, qpart, colv0, colv1, colv2, rowv0, rowv1, rowv2,
             rvs0, rvs1, rvs2, rows0, rows1, rows2, acc,
             gs0, gs1, gs2, ss0, ss1, ss2, is0, is1, is2):
    cid = lax.axis_index("c")
    sid = lax.axis_index("s")
    wid = cid * NS + sid
    zbase = sid * ROWS_PER_TILE
    pltpu.sync_copy(zero_hbm.at[pl.ds(0, ROWS_PER_TILE)],
                    acc.at[pl.ds(zbase, ROWS_PER_TILE)])
    pltpu.sync_copy(zq_hbm, qpart)
    pltpu.sync_copy(q_hbm.at[0], qtab)
    plsc.subcore_barrier()

    rows = (rows0, rows1, rows2)
    rowv = (rowv0, rowv1, rowv2)
    rvs = (rvs0, rvs1, rvs2)
    colv = (colv0, colv1, colv2)
    gsem = (gs0, gs1, gs2)
    ssem = (ss0, ss1, ss2)
    isem = (is0, is1, is2)

    def g_start(a):
        pltpu.async_copy(g_hbm.at[colv[a]], rows[a], gsem[a])

    def g_wait(a):
        pltpu.make_async_copy(g_hbm.at[colv[a]], rows[a], gsem[a]).wait()

    def i_start(c, a):
        pltpu.async_copy(col3_hbm.at[wid, c], colv[a], isem[a])
        pltpu.async_copy(row3_hbm.at[wid, c], rowv[a], isem[a])

    def i_wait(c, a):
        pltpu.make_async_copy(col3_hbm.at[wid, c], colv[a], isem[a]).wait()
        pltpu.make_async_copy(row3_hbm.at[wid, c], rowv[a], isem[a]).wait()

    def s_start(a):
        pltpu.async_copy(rows[a], acc.at[rvs[a]], ssem[a], add=True)

    def s_wait(a):
        pltpu.make_async_copy(rows[a], acc.at[rvs[a]], ssem[a]).wait()

    def rv_copy(a):
        for g in range(B // 16):
            rvs[a][pl.ds(g * 16, 16)] = rowv[a][pl.ds(g * 16, 16)]

    def q_accum(a):
        for g in range(B // 16):
            cv = colv[a][pl.ds(g * 16, 16)]
            rv = rowv[a][pl.ds(g * 16, 16)]
            qe = plsc.load_gather(qtab, [cv])
            plsc.addupdate_scatter(qpart, [rv], qe)

    i_start(0, 0)
    i_start(1, 1)
    i_wait(0, 0)
    rv_copy(0)
    g_start(0)
    i_wait(1, 1)
    rv_copy(1)
    g_start(1)
    i_start(2, 2)

    NITER = (NCHUNK - 1) // 3

    def body(j, carry):
        c0 = 3 * j
        for k in range(3):
            c = c0 + k
            a = k % 3
            prev = (k + 2) % 3
            g_wait(a)
            s_start(a)
            q_accum(a)
            if k == 0:
                @pl.when(j > 0)
                def _():
                    s_wait(prev)
                i_wait(c + 2, prev)
                rv_copy(prev)
                i_start(c + 3, a)
                g_start(prev)
            elif k == 1:
                s_wait(prev)
                i_wait(c + 2, prev)
                rv_copy(prev)

                @pl.when(j < NITER - 1)
                def _():
                    i_start(c + 3, a)
                g_start(prev)
            else:
                s_wait(prev)

                @pl.when(j < NITER - 1)
                def _():
                    i_wait(c + 2, prev)
                    rv_copy(prev)
                    i_start(c + 3, a)
                    g_start(prev)
        return carry

    lax.fori_loop(0, NITER, body, 0)
    c_last = NCHUNK - 1
    a_last = c_last % 3
    g_wait(a_last)
    s_start(a_last)
    q_accum(a_last)
    s_wait((c_last + 2) % 3)
    s_wait(a_last)
    plsc.subcore_barrier()

    src = acc.at[pl.ds(zbase, ROWS_PER_TILE)]

    @pl.when(cid == 0)
    def _():
        pltpu.sync_copy(src, out0_hbm.at[pl.ds(zbase, ROWS_PER_TILE)])

    @pl.when(cid == 1)
    def _():
        pltpu.sync_copy(src, out1_hbm.at[pl.ds(zbase, ROWS_PER_TILE)])

    pltpu.sync_copy(qpart, qp_hbm.at[wid])


def kernel(features, adj_nei, high_att_0, diff_att_0):
    g, qt = pl.pallas_call(
        _prologue_body,
        out_shape=(jax.ShapeDtypeStruct((NODE, D), jnp.float32),
                   jax.ShapeDtypeStruct((1, NODE), jnp.float32)),
    )(features, high_att_0, diff_att_0)

    pad_i = jnp.arange(EPAD, dtype=jnp.int32)
    row3 = jnp.concatenate(
        [adj_nei[0], NODE + pad_i % (NP - NODE)]).reshape(NW, NCHUNK, B)
    col3 = jnp.concatenate(
        [adj_nei[1], pad_i % NODE]).reshape(NW, NCHUNK, B)
    zeros = jnp.zeros((ROWS_PER_TILE, D), jnp.float32)
    zq = jnp.zeros((NP,), jnp.float32)

    sc_fn = functools.partial(
        pl.kernel,
        mesh=plsc.VectorSubcoreMesh(core_axis_name="c", subcore_axis_name="s"),
        out_type=(jax.ShapeDtypeStruct((NP, D), jnp.float32),
                  jax.ShapeDtypeStruct((NP, D), jnp.float32),
                  jax.ShapeDtypeStruct((NW, NP), jnp.float32)),
        scratch_types=(
            [pltpu.VMEM((NODE,), jnp.float32),
             pltpu.VMEM((NP,), jnp.float32)]
            + [pltpu.VMEM((B,), jnp.int32) for _ in range(9)]
            + [pltpu.VMEM((B, D), jnp.float32) for _ in range(3)]
            + [pltpu.VMEM_SHARED((NP, D), jnp.float32)]
            + [pltpu.SemaphoreType.DMA for _ in range(9)]
        ),
        compiler_params=pltpu.CompilerParams(needs_layout_passes=False),
    )(_sc_body)
    hp0, hp1, qparts = sc_fn(g, qt, row3, col3, zeros, zq)

    out = pl.pallas_call(
        _epilogue_body,
        out_shape=jax.ShapeDtypeStruct((NODE, D), jnp.float32),
    )(hp0, hp1, qparts)
    return out
